# baseline (device time: 22643 ns/iter reference)
import jax
import jax.numpy as jnp
from jax import lax
from jax.experimental import pallas as pl
from jax.experimental.pallas import tpu as pltpu

N_DEV = 4
B = 2
SQ = 128
SKV = 128
SV = 2 * SKV
HQ = 4
DH = 64
DM = 512
HD = HQ * DH
NEED = 2
QR = (B * SQ) // N_DEV


def kernel(x, Wq, K_ext, V_ext, Wo):
    x2 = x.reshape(B * SQ, DM).astype(jnp.bfloat16)
    wq_b = Wq.astype(jnp.bfloat16)
    wo_b = Wo.astype(jnp.bfloat16)
    k2 = K_ext.reshape(B, SKV, 16 * DH).astype(jnp.bfloat16)
    v2 = V_ext.reshape(B, SKV, 16 * DH).astype(jnp.bfloat16)

    def body(x_ref, wq_ref, k_ref, v_ref, wo_ref, out_ref,
             kv_buf, part_buf, rs_buf, ag_buf,
             p1_send, p1_recv, rs_send, rs_recv, ag_send, ag_recv):
        me = lax.axis_index("i")

        bsem = pltpu.get_barrier_semaphore()
        for k in range(1, N_DEV):
            dst = lax.rem(me + k, N_DEV)
            pl.semaphore_signal(bsem, inc=1, device_id=(dst,),
                                device_id_type=pl.DeviceIdType.MESH)
        pl.semaphore_wait(bsem, N_DEV - 1)

        def p1_recv_desc(kvi, blk):
            return pltpu.make_async_remote_copy(
                src_ref=kv_buf.at[kvi, :, pl.ds(blk * SKV, SKV), :],
                dst_ref=kv_buf.at[kvi, :, pl.ds(blk * SKV, SKV), :],
                send_sem=p1_send.at[0],
                recv_sem=p1_recv.at[kvi, blk],
                device_id=(0,),
                device_id_type=pl.DeviceIdType.MESH,
            )

        def p1_send_descs(src):
            descs = []
            i = 0
            for kvi, ref in ((0, k_ref), (1, v_ref)):
                for dstd in range(N_DEV):
                    if dstd == src:
                        continue
                    descs.append(pltpu.make_async_remote_copy(
                        src_ref=ref.at[:, :, pl.ds(dstd * HD, HD)],
                        dst_ref=kv_buf.at[kvi, :, pl.ds(src * SKV, SKV), :],
                        send_sem=p1_send.at[i],
                        recv_sem=p1_recv.at[kvi, src],
                        device_id=(dstd,),
                        device_id_type=pl.DeviceIdType.MESH,
                    ))
                    i += 1
            return descs

        for src in range(NEED):
            @pl.when(me == src)
            def _(src=src):
                kv_buf[0, :, src * SKV:(src + 1) * SKV, :] = \
                    k_ref[:, :, src * HD:(src + 1) * HD]
                kv_buf[1, :, src * SKV:(src + 1) * SKV, :] = \
                    v_ref[:, :, src * HD:(src + 1) * HD]
                for d in p1_send_descs(src):
                    d.start()

        q = jnp.dot(x_ref[:], wq_ref[:],
                    preferred_element_type=jnp.float32) * 0.125
        q = q.astype(jnp.bfloat16)

        for blk in range(NEED):
            @pl.when(me != blk)
            def _(blk=blk):
                p1_recv_desc(0, blk).wait_recv()

        rows = lax.broadcasted_iota(jnp.int32, (SQ, SV), 0)
        cols = lax.broadcasted_iota(jnp.int32, (SQ, SV), 1)
        maskc = cols <= rows + SKV

        dn_t = (((1,), (1,)), ((), ()))
        wd = []
        for b in range(B):
            for h in range(HQ):
                qbh = q[b * SQ:(b + 1) * SQ, h * DH:(h + 1) * DH]
                kall = kv_buf[0, b, :, h * DH:(h + 1) * DH]
                s = lax.dot_general(qbh, kall, dn_t,
                                    preferred_element_type=jnp.float32)
                s = jnp.where(maskc, s, -1e9)
                m = jnp.max(s, axis=1, keepdims=True)
                w = jnp.exp(s - m)
                den = jnp.sum(w, axis=1, keepdims=True)
                wd.append((b, h, w.astype(jnp.bfloat16), den))

        for blk in range(NEED):
            @pl.when(me != blk)
            def _(blk=blk):
                p1_recv_desc(1, blk).wait_recv()

        parts = []
        hs = []
        for b, h, w, den in wd:
            vall = kv_buf[1, b, :, h * DH:(h + 1) * DH]
            ctxh = jnp.dot(w, vall, preferred_element_type=jnp.float32) / den
            hs.append(ctxh.astype(jnp.bfloat16))
            if h == HQ - 1:
                parts.append(jnp.concatenate(hs, axis=1))
                hs = []
        ctx_all = jnp.concatenate(parts, axis=0)
        partial = jnp.dot(ctx_all, wo_ref[:],
                          preferred_element_type=jnp.float32)
        partial = partial.astype(jnp.bfloat16)
        part_buf[:] = partial

        def rs_send_descs(src):
            descs = []
            i = 0
            for dstd in range(N_DEV):
                if dstd == src:
                    continue
                descs.append(pltpu.make_async_remote_copy(
                    src_ref=part_buf.at[pl.ds(dstd * QR, QR), :],
                    dst_ref=rs_buf.at[src],
                    send_sem=rs_send.at[i],
                    recv_sem=rs_recv.at[src],
                    device_id=(dstd,),
                    device_id_type=pl.DeviceIdType.MESH,
                ))
                i += 1
            return descs

        def ag_send_descs(src):
            descs = []
            i = 0
            for dstd in range(N_DEV):
                if dstd == src:
                    continue
                descs.append(pltpu.make_async_remote_copy(
                    src_ref=ag_buf.at[src],
                    dst_ref=ag_buf.at[src],
                    send_sem=ag_send.at[i],
                    recv_sem=ag_recv.at[src],
                    device_id=(dstd,),
                    device_id_type=pl.DeviceIdType.MESH,
                ))
                i += 1
            return descs

        def simple_recv(buf_slot, sem):
            return pltpu.make_async_remote_copy(
                src_ref=buf_slot, dst_ref=buf_slot,
                send_sem=rs_send.at[0], recv_sem=sem,
                device_id=(0,), device_id_type=pl.DeviceIdType.MESH,
            )

        for src in range(N_DEV):
            @pl.when(me == src)
            def _(src=src):
                rs_buf[src] = partial[src * QR:(src + 1) * QR, :]
                for d in rs_send_descs(src):
                    d.start()
        for src in range(N_DEV):
            @pl.when(me != src)
            def _(src=src):
                simple_recv(rs_buf.at[src], rs_recv.at[src]).wait_recv()
        red = (rs_buf[0].astype(jnp.float32) + rs_buf[1].astype(jnp.float32) +
               rs_buf[2].astype(jnp.float32) + rs_buf[3].astype(jnp.float32))

        for src in range(N_DEV):
            @pl.when(me == src)
            def _(src=src):
                ag_buf[src] = red.astype(jnp.bfloat16)
                for d in ag_send_descs(src):
                    d.start()
        for src in range(N_DEV):
            @pl.when(me != src)
            def _(src=src):
                simple_recv(ag_buf.at[src], ag_recv.at[src]).wait_recv()

        for s4 in range(N_DEV):
            out_ref[s4 // 2, (s4 % 2) * QR:(s4 % 2) * QR + QR, :] = \
                ag_buf[s4].astype(jnp.float32)

        for src in range(NEED):
            @pl.when(me == src)
            def _(src=src):
                for d in p1_send_descs(src):
                    d.wait_send()
        for src in range(N_DEV):
            @pl.when(me == src)
            def _(src=src):
                for d in rs_send_descs(src):
                    d.wait_send()
                for d in ag_send_descs(src):
                    d.wait_send()

    return pl.pallas_call(
        body,
        out_shape=jax.ShapeDtypeStruct((B, SQ, DM), jnp.float32),
        in_specs=[pl.BlockSpec(memory_space=pltpu.VMEM)] * 5,
        out_specs=pl.BlockSpec(memory_space=pltpu.VMEM),
        scratch_shapes=[
            pltpu.VMEM((2, B, SV, HD), jnp.bfloat16),
            pltpu.VMEM((B * SQ, DM), jnp.bfloat16),
            pltpu.VMEM((N_DEV, QR, DM), jnp.bfloat16),
            pltpu.VMEM((N_DEV, QR, DM), jnp.bfloat16),
            pltpu.SemaphoreType.DMA((2 * (N_DEV - 1),)),
            pltpu.SemaphoreType.DMA((2, NEED)),
            pltpu.SemaphoreType.DMA((N_DEV - 1,)),
            pltpu.SemaphoreType.DMA((N_DEV,)),
            pltpu.SemaphoreType.DMA((N_DEV - 1,)),
            pltpu.SemaphoreType.DMA((N_DEV,)),
        ],
        compiler_params=pltpu.CompilerParams(collective_id=0),
    )(x2, wq_b, k2, v2, wo_b)


# device time: 20667 ns/iter; 1.0956x vs baseline; 1.0956x over previous
import jax
import jax.numpy as jnp
from jax import lax
from jax.experimental import pallas as pl
from jax.experimental.pallas import tpu as pltpu

N_DEV = 4
B = 2
SQ = 128
SKV = 128
SV = 2 * SKV
HQ = 4
DH = 64
DM = 512
HD = HQ * DH
NEED = 2
QR = (B * SQ) // N_DEV


def kernel(x, Wq, K_ext, V_ext, Wo):
    x2 = x.reshape(B * SQ, DM)
    k2 = K_ext.reshape(B, SKV, 16 * DH)
    v2 = V_ext.reshape(B, SKV, 16 * DH)

    def body(x_ref, wq_ref, k_ref, v_ref, wo_ref, out_ref,
             kv_buf, kv_src, kv_send, part_buf, rs_buf, ag_buf,
             cp_sem, p1_send, p1_recv, rs_send, rs_recv, ag_send, ag_recv):
        me = lax.axis_index("i")

        for src in range(NEED):
            @pl.when(me == src)
            def _():
                pltpu.make_async_copy(k_ref, kv_src.at[0], cp_sem.at[0]).start()
                pltpu.make_async_copy(v_ref, kv_src.at[1], cp_sem.at[1]).start()

        bsem = pltpu.get_barrier_semaphore()
        for k in range(1, N_DEV):
            dst = lax.rem(me + k, N_DEV)
            pl.semaphore_signal(bsem, inc=1, device_id=(dst,),
                                device_id_type=pl.DeviceIdType.MESH)
        pl.semaphore_wait(bsem, N_DEV - 1)

        def p1_recv_desc(kvi, blk):
            return pltpu.make_async_remote_copy(
                src_ref=kv_buf.at[kvi, :, pl.ds(blk * SKV, SKV), :],
                dst_ref=kv_buf.at[kvi, :, pl.ds(blk * SKV, SKV), :],
                send_sem=p1_send.at[0],
                recv_sem=p1_recv.at[kvi, blk],
                device_id=(0,),
                device_id_type=pl.DeviceIdType.MESH,
            )

        def p1_send_descs(src, kvi):
            descs = []
            for j, dstd in enumerate(d for d in range(N_DEV) if d != src):
                descs.append(pltpu.make_async_remote_copy(
                    src_ref=kv_send.at[kvi, :, :, pl.ds(dstd * HD, HD)],
                    dst_ref=kv_buf.at[kvi, :, pl.ds(src * SKV, SKV), :],
                    send_sem=p1_send.at[kvi * (N_DEV - 1) + j],
                    recv_sem=p1_recv.at[kvi, src],
                    device_id=(dstd,),
                    device_id_type=pl.DeviceIdType.MESH,
                ))
            return descs

        for src in range(NEED):
            @pl.when(me == src)
            def _(src=src):
                pltpu.make_async_copy(k_ref, kv_src.at[0], cp_sem.at[0]).wait()
                kv_send[0] = kv_src[0].astype(jnp.bfloat16)
                for d in p1_send_descs(src, 0):
                    d.start()
                pltpu.make_async_copy(v_ref, kv_src.at[1], cp_sem.at[1]).wait()
                kv_send[1] = kv_src[1].astype(jnp.bfloat16)
                for d in p1_send_descs(src, 1):
                    d.start()
                kv_buf[0, :, src * SKV:(src + 1) * SKV, :] = \
                    kv_send[0, :, :, src * HD:(src + 1) * HD]
                kv_buf[1, :, src * SKV:(src + 1) * SKV, :] = \
                    kv_send[1, :, :, src * HD:(src + 1) * HD]

        q = jnp.dot(x_ref[:], wq_ref[:],
                    preferred_element_type=jnp.float32) * 0.125

        for blk in range(NEED):
            @pl.when(me != blk)
            def _(blk=blk):
                p1_recv_desc(0, blk).wait_recv()

        rows = lax.broadcasted_iota(jnp.int32, (SQ, SV), 0)
        cols = lax.broadcasted_iota(jnp.int32, (SQ, SV), 1)
        maskc = cols <= rows + SKV

        dn_t = (((1,), (1,)), ((), ()))
        wd = {}
        for b in range(B):
            for h in range(HQ):
                qbh = q[b * SQ:(b + 1) * SQ, h * DH:(h + 1) * DH]
                kall = kv_buf[0, b, :, h * DH:(h + 1) * DH].astype(jnp.float32)
                s = lax.dot_general(qbh, kall, dn_t,
                                    preferred_element_type=jnp.float32)
                s = jnp.where(maskc, s, -1e9)
                w = jnp.exp(s)
                den = jnp.sum(w, axis=1, keepdims=True)
                wd[(b, h)] = (w, den)

        for blk in range(NEED):
            @pl.when(me != blk)
            def _(blk=blk):
                p1_recv_desc(1, blk).wait_recv()

        parts = []
        hs = []
        for b in range(B):
            for h in range(HQ):
                w, den = wd[(b, h)]
                vall = kv_buf[1, b, :, h * DH:(h + 1) * DH].astype(jnp.float32)
                hs.append(jnp.dot(w, vall,
                                  preferred_element_type=jnp.float32) / den)
            parts.append(jnp.concatenate(hs, axis=1))
            hs = []
        ctx_all = jnp.concatenate(parts, axis=0)
        partial = jnp.dot(ctx_all, wo_ref[:],
                          preferred_element_type=jnp.float32)
        partial = partial.astype(jnp.bfloat16)

        def rs_send_desc(src, t):
            sem_i = sum(1 for tt in range(t) if tt != src)
            return pltpu.make_async_remote_copy(
                src_ref=part_buf.at[pl.ds(t * QR, QR), :],
                dst_ref=rs_buf.at[src],
                send_sem=rs_send.at[sem_i],
                recv_sem=rs_recv.at[src],
                device_id=(t,),
                device_id_type=pl.DeviceIdType.MESH,
            )

        for src in range(N_DEV):
            @pl.when(me == src)
            def _(src=src):
                part_buf[:] = partial
                for t in range(N_DEV):
                    if t != src:
                        rs_send_desc(src, t).start()
                rs_buf[src] = partial[src * QR:(src + 1) * QR, :]

        for src in range(N_DEV):
            @pl.when(me != src)
            def _(src=src):
                pltpu.make_async_remote_copy(
                    src_ref=rs_buf.at[src], dst_ref=rs_buf.at[src],
                    send_sem=rs_send.at[0], recv_sem=rs_recv.at[src],
                    device_id=(0,), device_id_type=pl.DeviceIdType.MESH,
                ).wait_recv()
        red = (rs_buf[0].astype(jnp.float32) + rs_buf[1].astype(jnp.float32) +
               rs_buf[2].astype(jnp.float32) + rs_buf[3].astype(jnp.float32))

        def ag_send_descs(src):
            descs = []
            i = 0
            for dstd in range(N_DEV):
                if dstd == src:
                    continue
                descs.append(pltpu.make_async_remote_copy(
                    src_ref=ag_buf.at[src],
                    dst_ref=ag_buf.at[src],
                    send_sem=ag_send.at[i],
                    recv_sem=ag_recv.at[src],
                    device_id=(dstd,),
                    device_id_type=pl.DeviceIdType.MESH,
                ))
                i += 1
            return descs

        for src in range(N_DEV):
            @pl.when(me == src)
            def _(src=src):
                ag_buf[src] = red.astype(jnp.bfloat16)
                for d in ag_send_descs(src):
                    d.start()
                out_ref[src // 2, (src % 2) * QR:(src % 2) * QR + QR, :] = red
        for src in range(N_DEV):
            @pl.when(me != src)
            def _(src=src):
                pltpu.make_async_remote_copy(
                    src_ref=ag_buf.at[src], dst_ref=ag_buf.at[src],
                    send_sem=ag_send.at[0], recv_sem=ag_recv.at[src],
                    device_id=(0,), device_id_type=pl.DeviceIdType.MESH,
                ).wait_recv()
                out_ref[src // 2, (src % 2) * QR:(src % 2) * QR + QR, :] = \
                    ag_buf[src].astype(jnp.float32)

        for src in range(NEED):
            @pl.when(me == src)
            def _(src=src):
                for kvi in range(2):
                    for d in p1_send_descs(src, kvi):
                        d.wait_send()
        for src in range(N_DEV):
            @pl.when(me == src)
            def _(src=src):
                for t in range(N_DEV):
                    if t != src:
                        rs_send_desc(src, t).wait_send()
                for d in ag_send_descs(src):
                    d.wait_send()

    return pl.pallas_call(
        body,
        out_shape=jax.ShapeDtypeStruct((B, SQ, DM), jnp.float32),
        in_specs=[
            pl.BlockSpec(memory_space=pltpu.VMEM),
            pl.BlockSpec(memory_space=pltpu.VMEM),
            pl.BlockSpec(memory_space=pl.ANY),
            pl.BlockSpec(memory_space=pl.ANY),
            pl.BlockSpec(memory_space=pltpu.VMEM),
        ],
        out_specs=pl.BlockSpec(memory_space=pltpu.VMEM),
        scratch_shapes=[
            pltpu.VMEM((2, B, SV, HD), jnp.bfloat16),
            pltpu.VMEM((2, B, SKV, 16 * DH), jnp.float32),
            pltpu.VMEM((2, B, SKV, 16 * DH), jnp.bfloat16),
            pltpu.VMEM((B * SQ, DM), jnp.bfloat16),
            pltpu.VMEM((N_DEV, QR, DM), jnp.bfloat16),
            pltpu.VMEM((N_DEV, QR, DM), jnp.bfloat16),
            pltpu.SemaphoreType.DMA((2,)),
            pltpu.SemaphoreType.DMA((2 * (N_DEV - 1),)),
            pltpu.SemaphoreType.DMA((2, NEED)),
            pltpu.SemaphoreType.DMA((N_DEV - 1,)),
            pltpu.SemaphoreType.DMA((N_DEV,)),
            pltpu.SemaphoreType.DMA((N_DEV - 1,)),
            pltpu.SemaphoreType.DMA((N_DEV,)),
        ],
        compiler_params=pltpu.CompilerParams(collective_id=0),
    )(x2, Wq, k2, v2, Wo)
